# Initial kernel scaffold; baseline (speedup 1.0000x reference)
import jax
import jax.numpy as jnp
from jax.experimental import pallas as pl


def kernel(x, edge_index, W1, b1, W2, b2):
    raise NotImplementedError("write your pallas kernel here")


# speedup vs baseline: 29.5152x; 29.5152x over previous
import functools

import jax
import jax.numpy as jnp
from jax import lax
from jax.experimental import pallas as pl
from jax.experimental.pallas import tpu as pltpu
from jax.experimental.pallas import tpu_sc as plsc

N = 50000
E = 1600000
D = 16
NPG = 50
B = N // NPG
NUM_LAYERS = 4
CAT_D = NUM_LAYERS * D

NC = 2
NS = 16
LANES = 16

EPT = E // NS
EC = 2000
N_ECHUNK = EPT // EC
RC = 400
N_RCHUNK = N // RC


def _rsqrt16(v):
    i = lax.bitcast_convert_type(v, jnp.int32)
    i = jnp.int32(0x5F3759DF) - jnp.right_shift(i, jnp.int32(1))
    y = lax.bitcast_convert_type(i, jnp.float32)
    for _ in range(3):
        y = y * (jnp.float32(1.5) - jnp.float32(0.5) * v * y * y)
    return y


def _sc_body(x_hbm, src_hbm, dst_hbm,
             xcs_hbm, t_hbm, bbc_hbm, abbc_hbm,
             acc_sp,
             si_v, di_v, rows_v, zero_v,
             a_v, b_v, c_v, o1_v):
    cid = lax.axis_index("c")
    sid = lax.axis_index("s")
    work = cid == 0

    def _fill_ones(i, _):
        rows_v[i, :] = jnp.full((LANES,), 1.0, jnp.float32)
        return 0
    lax.fori_loop(0, EC, _fill_ones, 0)

    def _fill_zero(i, _):
        zero_v[i, :] = jnp.zeros((LANES,), jnp.float32)
        return 0
    lax.fori_loop(0, RC, _fill_zero, 0)

    def _row_loop(fn):
        def body(j, _):
            k = j * NS + sid

            @pl.when(k < N_RCHUNK)
            def _():
                fn(k * RC)
            return 0
        lax.fori_loop(0, (N_RCHUNK + NS - 1) // NS, body, 0)

    @pl.when(work)
    def _():
        def z(r0):
            pltpu.sync_copy(zero_v, acc_sp.at[pl.ds(r0, RC)])
        _row_loop(z)

    plsc.subcore_barrier()

    @pl.when(work)
    def _():
        def body(c, _):
            base = sid * EPT + c * EC
            pltpu.sync_copy(src_hbm.at[pl.ds(base, EC)], si_v)
            pltpu.sync_copy(rows_v, acc_sp.at[si_v], add=True)
            return 0
        lax.fori_loop(0, N_ECHUNK, body, 0)

    plsc.subcore_barrier()

    @pl.when(work)
    def _():
        def body(r0):
            pltpu.sync_copy(acc_sp.at[pl.ds(r0, RC)], a_v)

            def rb(i, _):
                o1_v[i, :] = _rsqrt16(jnp.maximum(a_v[i, :], jnp.float32(1.0)))
                return 0
            lax.fori_loop(0, RC, rb, 0)
            pltpu.sync_copy(o1_v, abbc_hbm.at[pl.ds(r0, RC)])
            pltpu.sync_copy(zero_v, acc_sp.at[pl.ds(r0, RC)])
        _row_loop(body)

    plsc.subcore_barrier()

    @pl.when(work)
    def _():
        def body(c, _):
            base = sid * EPT + c * EC
            pltpu.sync_copy(dst_hbm.at[pl.ds(base, EC)], di_v)
            pltpu.sync_copy(rows_v, acc_sp.at[di_v], add=True)
            return 0
        lax.fori_loop(0, N_ECHUNK, body, 0)

    plsc.subcore_barrier()

    @pl.when(work)
    def _():
        def body(r0):
            pltpu.sync_copy(acc_sp.at[pl.ds(r0, RC)], a_v)
            pltpu.sync_copy(abbc_hbm.at[pl.ds(r0, RC)], b_v)
            pltpu.sync_copy(x_hbm.at[pl.ds(r0, RC)], c_v)

            def rb(i, _):
                b = _rsqrt16(jnp.maximum(a_v[i, :], jnp.float32(1.0)))
                a = b_v[i, :]
                o1_v[i, :] = b
                a_v[i, :] = a * b
                c_v[i, :] = c_v[i, :] * a
                return 0
            lax.fori_loop(0, RC, rb, 0)
            pltpu.sync_copy(o1_v, bbc_hbm.at[pl.ds(r0, RC)])
            pltpu.sync_copy(a_v, abbc_hbm.at[pl.ds(r0, RC)])
            pltpu.sync_copy(c_v, t_hbm.at[pl.ds(r0, RC)])
            pltpu.sync_copy(zero_v, acc_sp.at[pl.ds(r0, RC)])
        _row_loop(body)

    plsc.subcore_barrier()

    for l in range(NUM_LAYERS):
        @pl.when(work)
        def _():
            def body(c, _):
                base = sid * EPT + c * EC
                pltpu.sync_copy(src_hbm.at[pl.ds(base, EC)], si_v)
                pltpu.sync_copy(dst_hbm.at[pl.ds(base, EC)], di_v)
                pltpu.sync_copy(t_hbm.at[si_v], rows_v)
                pltpu.sync_copy(rows_v, acc_sp.at[di_v], add=True)
                return 0
            lax.fori_loop(0, N_ECHUNK, body, 0)

        plsc.subcore_barrier()

        @pl.when(work)
        def _(l=l):
            def body(r0):
                pltpu.sync_copy(acc_sp.at[pl.ds(r0, RC)], a_v)
                pltpu.sync_copy(bbc_hbm.at[pl.ds(r0, RC)], b_v)
                pltpu.sync_copy(abbc_hbm.at[pl.ds(r0, RC)], c_v)

                def rb(i, _):
                    acc = a_v[i, :]
                    o1_v[i, :] = acc * b_v[i, :]
                    a_v[i, :] = acc * c_v[i, :]
                    return 0
                lax.fori_loop(0, RC, rb, 0)
                pltpu.sync_copy(o1_v, xcs_hbm.at[l, pl.ds(r0, RC)])
                if l < NUM_LAYERS - 1:
                    pltpu.sync_copy(a_v, t_hbm.at[pl.ds(r0, RC)])
                    pltpu.sync_copy(zero_v, acc_sp.at[pl.ds(r0, RC)])
            _row_loop(body)

        plsc.subcore_barrier()


@functools.cache
def _make_graph_kernel():
    mesh = plsc.VectorSubcoreMesh(
        core_axis_name="c", subcore_axis_name="s",
        num_cores=NC, num_subcores=NS)
    return pl.kernel(
        _sc_body,
        out_type=(
            jax.ShapeDtypeStruct((NUM_LAYERS, N, D), jnp.float32),
            jax.ShapeDtypeStruct((N, D), jnp.float32),
            jax.ShapeDtypeStruct((N, D), jnp.float32),
            jax.ShapeDtypeStruct((N, D), jnp.float32),
        ),
        mesh=mesh,
        scratch_types=[
            pltpu.VMEM_SHARED((N, D), jnp.float32),
            pltpu.VMEM((EC,), jnp.int32),
            pltpu.VMEM((EC,), jnp.int32),
            pltpu.VMEM((EC, D), jnp.float32),
            pltpu.VMEM((RC, D), jnp.float32),
            pltpu.VMEM((RC, D), jnp.float32),
            pltpu.VMEM((RC, D), jnp.float32),
            pltpu.VMEM((RC, D), jnp.float32),
            pltpu.VMEM((RC, D), jnp.float32),
        ],
        compiler_params=pltpu.CompilerParams(use_tc_tiling_on_sc=False),
    )


def _head_body(xcs_ref, w1_ref, b1_ref, w2_ref, b2_ref, out_ref):
    xb = xcs_ref[...]
    xg = jnp.concatenate([xb[0], xb[1], xb[2], xb[3]], axis=-1)
    scale = jnp.float32(1.0) / jnp.sqrt(jnp.float32(CAT_D))
    half = NPG // 2
    users = xg[:, :half, :]
    items = xg[:, half:, :]
    q_u = xg[:, 0, :]
    q_i = xg[:, half, :]

    def pool(seg, q):
        s = jnp.sum(seg * q[:, None, :], axis=-1) * scale
        m = jnp.max(s, axis=1, keepdims=True)
        e = jnp.exp(s - m)
        a = e / jnp.sum(e, axis=1, keepdims=True)
        return jnp.sum(a[:, :, None] * seg, axis=1)

    z = jnp.concatenate([pool(users, q_u), pool(items, q_i)], axis=-1)
    h = jnp.maximum(jnp.dot(z, w1_ref[...],
                            preferred_element_type=jnp.float32)
                    + b1_ref[...], 0.0)
    o = jnp.sum(h * w2_ref[...], axis=-1, keepdims=True) + b2_ref[...]
    o = jnp.float32(1.0) / (jnp.float32(1.0) + jnp.exp(-o))
    out_ref[...] = jnp.broadcast_to(o, (o.shape[0], 128))


_G = 40

_head_call = pl.pallas_call(
    _head_body,
    grid=(B // _G,),
    in_specs=[
        pl.BlockSpec((NUM_LAYERS, _G, NPG, D), lambda i: (0, i, 0, 0)),
        pl.BlockSpec((2 * CAT_D, 64), lambda i: (0, 0)),
        pl.BlockSpec((1, 64), lambda i: (0, 0)),
        pl.BlockSpec((1, 64), lambda i: (0, 0)),
        pl.BlockSpec((1, 1), lambda i: (0, 0)),
    ],
    out_specs=pl.BlockSpec((_G, 128), lambda i: (i, 0)),
    out_shape=jax.ShapeDtypeStruct((B, 128), jnp.float32),
)


def kernel(x, edge_index, W1, b1, W2, b2):
    src = edge_index[0]
    dst = edge_index[1]
    xcs, _t, _bb, _ab = _make_graph_kernel()(x, src, dst)
    xcs4 = xcs.reshape(NUM_LAYERS, B, NPG, D)
    out2d = _head_call(xcs4, W1, b1.reshape(1, 64),
                       W2.reshape(1, 64), b2.reshape(1, 1))
    return out2d[:, 0]
